# baseline (speedup 1.0000x reference)
import jax
import jax.numpy as jnp
from jax.experimental import pallas as pl
from jax.experimental.pallas import tpu as pltpu

BN_EPS = 1e-5
NV = 20000
P = 35
PP = 40
CIN = 7
D, H, W = 10, 200, 176
DHW = D * H * W
NROWS = NV * PP
NREAL = NV * P
VB = 200
BLKR = VB * PP
NBLK = NV // VB
NSPLIT = 16
SPLIT = DHW // NSPLIT

_f32 = jnp.float32


def _valid_mask():
    return (jax.lax.broadcasted_iota(jnp.int32, (VB, PP, 1), 1) < P).astype(_f32)


def _keep(xb):
    return (jnp.sum(xb, axis=1, keepdims=True) != 0.0).astype(_f32)


def _vfe_stage(a, keep):
    c = a.shape[1]
    a3 = a.reshape(VB, PP, c)
    m = jnp.max(a3 * _valid_mask(), axis=1, keepdims=True)
    cat = jnp.concatenate([a3, jnp.broadcast_to(m, a3.shape)], axis=-1)
    return cat.reshape(BLKR, 2 * c) * keep


def _k1(x_ref, s_ref, ss_ref):
    xb = x_ref[...]
    s_ref[...] = jnp.sum(xb, axis=0).reshape(1, 1, CIN)
    ss_ref[...] = jax.lax.dot_general(
        xb, xb, (((0,), (0,)), ((), ())), preferred_element_type=_f32, precision=jax.lax.Precision.HIGHEST
    ).reshape(1, CIN, CIN)


def _h1(xb, w1_ref, c1_ref):
    keep = _keep(xb)
    a1 = jnp.maximum(
        jnp.dot(xb, w1_ref[...], preferred_element_type=_f32) + c1_ref[...], 0.0)
    return _vfe_stage(a1, keep), keep


def _k2(x_ref, w1_ref, c1_ref, s_ref, ss_ref):
    h1, _ = _h1(x_ref[...], w1_ref, c1_ref)
    s_ref[...] = jnp.sum(h1, axis=0).reshape(1, 1, 32)
    ss_ref[...] = jax.lax.dot_general(
        h1, h1, (((0,), (0,)), ((), ())), preferred_element_type=_f32, precision=jax.lax.Precision.HIGHEST
    ).reshape(1, 32, 32)


def _h2(xb, w1_ref, c1_ref, w2_ref, c2_ref):
    h1, keep = _h1(xb, w1_ref, c1_ref)
    a2 = jnp.maximum(
        jnp.dot(h1, w2_ref[...], preferred_element_type=_f32) + c2_ref[...], 0.0)
    return _vfe_stage(a2, keep)


def _k3(x_ref, w1_ref, c1_ref, w2_ref, c2_ref, s_ref, ss_ref):
    h2 = _h2(x_ref[...], w1_ref, c1_ref, w2_ref, c2_ref)
    s_ref[...] = jnp.sum(h2, axis=0).reshape(1, 1, 128)
    ss_ref[...] = jax.lax.dot_general(
        h2, h2, (((0,), (0,)), ((), ())), preferred_element_type=_f32, precision=jax.lax.Precision.HIGHEST
    ).reshape(1, 128, 128)


def _k4(x_ref, w1_ref, c1_ref, w2_ref, c2_ref, wd_ref, cd_ref, f_ref):
    h2 = _h2(x_ref[...], w1_ref, c1_ref, w2_ref, c2_ref)
    ad = jnp.maximum(
        jnp.dot(h2, wd_ref[...], preferred_element_type=_f32) + cd_ref[...], 0.0)
    ad3 = ad.reshape(VB, PP, 128) * _valid_mask()
    f_ref[...] = jnp.max(ad3, axis=1)


def _k5(idx_ref, f_ref, out_ref):
    s = pl.program_id(0)
    base = s * SPLIT
    out_ref[...] = jnp.zeros_like(out_ref)

    def body(v, carry):
        local = idx_ref[v] - base

        @pl.when((local >= 0) & (local < SPLIT))
        def _():
            out_ref[pl.ds(local, 1), :, :] += f_ref[pl.ds(v, 1), :, :]

        return carry

    jax.lax.fori_loop(0, NV, body, 0)


def _moment_pass(body, x2d, extras, cout):
    params = pltpu.CompilerParams(
        dimension_semantics=("parallel",), vmem_limit_bytes=100 * 1024 * 1024)
    s_p, ss_p = pl.pallas_call(
        body,
        grid=(NBLK,),
        in_specs=[pl.BlockSpec((BLKR, CIN), lambda i: (i, 0))]
        + [pl.BlockSpec(e.shape, lambda i: (0,) * e.ndim) for e in extras],
        out_specs=[
            pl.BlockSpec((1, 1, cout), lambda i: (i, 0, 0)),
            pl.BlockSpec((1, cout, cout), lambda i: (i, 0, 0)),
        ],
        out_shape=[
            jax.ShapeDtypeStruct((NBLK, 1, cout), _f32),
            jax.ShapeDtypeStruct((NBLK, cout, cout), _f32),
        ],
        compiler_params=params,
    )(x2d, *extras)
    return s_p.sum(axis=(0, 1)), ss_p.sum(axis=0)


def _bn_consts(S, SS, Wl, bl, g, be):
    mu = S / NREAL
    m2 = SS / NREAL
    hi = jax.lax.Precision.HIGHEST
    muW = jnp.dot(mu, Wl, precision=hi)
    mean = muW + bl
    second = (jnp.einsum('cu,cd,du->u', Wl, m2, Wl, precision=hi)
              + 2.0 * bl * muW + bl * bl)
    var = second - mean * mean
    scale = g * jax.lax.rsqrt(var + BN_EPS)
    shift = be - mean * scale
    return Wl * scale[None, :], (bl * scale + shift).reshape(1, -1)


def kernel(x, coordinate, W1, b1, g1, be1, W2, b2, g2, be2, Wd, bd, gd, bed):
    xp = jnp.pad(x[0], ((0, 0), (0, PP - P), (0, 0))).reshape(NROWS, CIN)
    idx = (coordinate[0, :, 0] * (H * W) + coordinate[0, :, 1] * W
           + coordinate[0, :, 2]).astype(jnp.int32)

    Sx, Sxx = _moment_pass(_k1, xp, [], CIN)
    W1p, c1 = _bn_consts(Sx, Sxx, W1, b1, g1, be1)

    Sh, Shh = _moment_pass(_k2, xp, [W1p, c1], 32)
    W2p, c2 = _bn_consts(Sh, Shh, W2, b2, g2, be2)

    Sg, Sgg = _moment_pass(_k3, xp, [W1p, c1, W2p, c2], 128)
    Wdp, cd = _bn_consts(Sg, Sgg, Wd, bd, gd, bed)

    extras = [W1p, c1, W2p, c2, Wdp, cd]
    feats = pl.pallas_call(
        _k4,
        grid=(NBLK,),
        in_specs=[pl.BlockSpec((BLKR, CIN), lambda i: (i, 0))]
        + [pl.BlockSpec(e.shape, lambda i: (0,) * e.ndim) for e in extras],
        out_specs=pl.BlockSpec((VB, 128), lambda i: (i, 0)),
        out_shape=jax.ShapeDtypeStruct((NV, 128), _f32),
        compiler_params=pltpu.CompilerParams(
            dimension_semantics=("parallel",),
            vmem_limit_bytes=100 * 1024 * 1024),
    )(xp, *extras)

    grid2d = pl.pallas_call(
        _k5,
        grid_spec=pltpu.PrefetchScalarGridSpec(
            num_scalar_prefetch=1,
            grid=(NSPLIT,),
            in_specs=[pl.BlockSpec((NV, 1, 128), lambda s, *_: (0, 0, 0))],
            out_specs=pl.BlockSpec((SPLIT, 1, 128), lambda s, *_: (s, 0, 0)),
        ),
        out_shape=jax.ShapeDtypeStruct((DHW, 1, 128), _f32),
        compiler_params=pltpu.CompilerParams(
            dimension_semantics=("parallel",),
            vmem_limit_bytes=100 * 1024 * 1024),
    )(idx, feats.reshape(NV, 1, 128))

    return grid2d.reshape(DHW, 128).T.reshape(1, 128, D, H, W)


# speedup vs baseline: 2.2172x; 2.2172x over previous
import jax
import jax.numpy as jnp
from jax.experimental import pallas as pl
from jax.experimental.pallas import tpu as pltpu

BN_EPS = 1e-5
NV = 20000
P = 35
PP = 40
CIN = 7
D, H, W = 10, 200, 176
DHW = D * H * W
NROWS = NV * PP
NREAL = NV * P
VB = 200
BLKR = VB * PP
NBLK = NV // VB
NSPLIT = 16
SPLIT = DHW // NSPLIT

_f32 = jnp.float32


def _valid_mask():
    return (jax.lax.broadcasted_iota(jnp.int32, (VB, PP, 1), 1) < P).astype(_f32)


def _keep(xb):
    return (jnp.sum(xb, axis=1, keepdims=True) != 0.0).astype(_f32)


def _vfe_stage(a, keep):
    c = a.shape[1]
    a3 = a.reshape(VB, PP, c)
    m = jnp.max(a3 * _valid_mask(), axis=1, keepdims=True)
    cat = jnp.concatenate([a3, jnp.broadcast_to(m, a3.shape)], axis=-1)
    return cat.reshape(BLKR, 2 * c) * keep


def _k1(x_ref, s_ref, ss_ref):
    xb = x_ref[...]
    s_ref[...] = jnp.sum(xb, axis=0).reshape(1, 1, CIN)
    ss_ref[...] = jax.lax.dot_general(
        xb, xb, (((0,), (0,)), ((), ())), preferred_element_type=_f32, precision=jax.lax.Precision.HIGHEST
    ).reshape(1, CIN, CIN)


def _h1(xb, w1_ref, c1_ref):
    keep = _keep(xb)
    a1 = jnp.maximum(
        jnp.dot(xb, w1_ref[...], preferred_element_type=_f32) + c1_ref[...], 0.0)
    return _vfe_stage(a1, keep), keep


def _k2(x_ref, w1_ref, c1_ref, s_ref, ss_ref):
    h1, _ = _h1(x_ref[...], w1_ref, c1_ref)
    s_ref[...] = jnp.sum(h1, axis=0).reshape(1, 1, 32)
    ss_ref[...] = jax.lax.dot_general(
        h1, h1, (((0,), (0,)), ((), ())), preferred_element_type=_f32, precision=jax.lax.Precision.HIGHEST
    ).reshape(1, 32, 32)


def _h2(xb, w1_ref, c1_ref, w2_ref, c2_ref):
    h1, keep = _h1(xb, w1_ref, c1_ref)
    a2 = jnp.maximum(
        jnp.dot(h1, w2_ref[...], preferred_element_type=_f32) + c2_ref[...], 0.0)
    return _vfe_stage(a2, keep)


def _k3(x_ref, w1_ref, c1_ref, w2_ref, c2_ref, s_ref, ss_ref):
    h2 = _h2(x_ref[...], w1_ref, c1_ref, w2_ref, c2_ref)
    s_ref[...] = jnp.sum(h2, axis=0).reshape(1, 1, 128)
    ss_ref[...] = jax.lax.dot_general(
        h2, h2, (((0,), (0,)), ((), ())), preferred_element_type=_f32, precision=jax.lax.Precision.HIGHEST
    ).reshape(1, 128, 128)


def _k4(x_ref, w1_ref, c1_ref, w2_ref, c2_ref, wd_ref, cd_ref, f_ref):
    h2 = _h2(x_ref[...], w1_ref, c1_ref, w2_ref, c2_ref)
    ad = jnp.maximum(
        jnp.dot(h2, wd_ref[...], preferred_element_type=_f32) + cd_ref[...], 0.0)
    ad3 = ad.reshape(VB, PP, 128) * _valid_mask()
    f_ref[...] = jnp.max(ad3, axis=1)


def _k5a(sid_ref, perm_ref, bases_ref, cnt_ref, off_ref):
    for i in range(NSPLIT):
        cnt_ref[i] = 0

    def count(v, c):
        s = sid_ref[v]
        cnt_ref[s] += 1
        return c

    jax.lax.fori_loop(0, NV, count, 0)
    run = 0
    for i in range(NSPLIT):
        bases_ref[i] = run
        off_ref[i] = run
        run = run + cnt_ref[i]
    bases_ref[NSPLIT] = run

    def place(v, c):
        s = sid_ref[v]
        pos = off_ref[s]
        off_ref[s] = pos + 1
        perm_ref[pos] = v
        return c

    jax.lax.fori_loop(0, NV, place, 0)


def _k5b(idx_ref, perm_ref, bases_ref, f_ref, out_ref):
    s = pl.program_id(0)
    base = s * SPLIT
    out_ref[...] = jnp.zeros_like(out_ref)

    def body(v, carry):
        w = perm_ref[v]
        local = idx_ref[w] - base
        out_ref[pl.ds(local, 1), :, :] += f_ref[pl.ds(w, 1), :, :]
        return carry

    jax.lax.fori_loop(bases_ref[s], bases_ref[s + 1], body, 0)


def _moment_pass(body, x2d, extras, cout):
    params = pltpu.CompilerParams(
        dimension_semantics=("parallel",), vmem_limit_bytes=100 * 1024 * 1024)
    s_p, ss_p = pl.pallas_call(
        body,
        grid=(NBLK,),
        in_specs=[pl.BlockSpec((BLKR, CIN), lambda i: (i, 0))]
        + [pl.BlockSpec(e.shape, lambda i: (0,) * e.ndim) for e in extras],
        out_specs=[
            pl.BlockSpec((1, 1, cout), lambda i: (i, 0, 0)),
            pl.BlockSpec((1, cout, cout), lambda i: (i, 0, 0)),
        ],
        out_shape=[
            jax.ShapeDtypeStruct((NBLK, 1, cout), _f32),
            jax.ShapeDtypeStruct((NBLK, cout, cout), _f32),
        ],
        compiler_params=params,
    )(x2d, *extras)
    return s_p.sum(axis=(0, 1)), ss_p.sum(axis=0)


def _bn_consts(S, SS, Wl, bl, g, be):
    mu = S / NREAL
    m2 = SS / NREAL
    hi = jax.lax.Precision.HIGHEST
    muW = jnp.dot(mu, Wl, precision=hi)
    mean = muW + bl
    second = (jnp.einsum('cu,cd,du->u', Wl, m2, Wl, precision=hi)
              + 2.0 * bl * muW + bl * bl)
    var = second - mean * mean
    scale = g * jax.lax.rsqrt(var + BN_EPS)
    shift = be - mean * scale
    return Wl * scale[None, :], (bl * scale + shift).reshape(1, -1)


def kernel(x, coordinate, W1, b1, g1, be1, W2, b2, g2, be2, Wd, bd, gd, bed):
    xp = jnp.pad(x[0], ((0, 0), (0, PP - P), (0, 0))).reshape(NROWS, CIN)
    idx = (coordinate[0, :, 0] * (H * W) + coordinate[0, :, 1] * W
           + coordinate[0, :, 2]).astype(jnp.int32)

    Sx, Sxx = _moment_pass(_k1, xp, [], CIN)
    W1p, c1 = _bn_consts(Sx, Sxx, W1, b1, g1, be1)

    Sh, Shh = _moment_pass(_k2, xp, [W1p, c1], 32)
    W2p, c2 = _bn_consts(Sh, Shh, W2, b2, g2, be2)

    Sg, Sgg = _moment_pass(_k3, xp, [W1p, c1, W2p, c2], 128)
    Wdp, cd = _bn_consts(Sg, Sgg, Wd, bd, gd, bed)

    extras = [W1p, c1, W2p, c2, Wdp, cd]
    feats = pl.pallas_call(
        _k4,
        grid=(NBLK,),
        in_specs=[pl.BlockSpec((BLKR, CIN), lambda i: (i, 0))]
        + [pl.BlockSpec(e.shape, lambda i: (0,) * e.ndim) for e in extras],
        out_specs=pl.BlockSpec((VB, 128), lambda i: (i, 0)),
        out_shape=jax.ShapeDtypeStruct((NV, 128), _f32),
        compiler_params=pltpu.CompilerParams(
            dimension_semantics=("parallel",),
            vmem_limit_bytes=100 * 1024 * 1024),
    )(xp, *extras)

    sid = idx // SPLIT
    perm, bases = pl.pallas_call(
        _k5a,
        grid_spec=pltpu.PrefetchScalarGridSpec(
            num_scalar_prefetch=1,
            grid=(1,),
            in_specs=[],
            out_specs=[
                pl.BlockSpec(memory_space=pltpu.SMEM),
                pl.BlockSpec(memory_space=pltpu.SMEM),
            ],
            scratch_shapes=[
                pltpu.SMEM((NSPLIT,), jnp.int32),
                pltpu.SMEM((NSPLIT,), jnp.int32),
            ],
        ),
        out_shape=[
            jax.ShapeDtypeStruct((NV,), jnp.int32),
            jax.ShapeDtypeStruct((NSPLIT + 1,), jnp.int32),
        ],
        compiler_params=pltpu.CompilerParams(
            vmem_limit_bytes=100 * 1024 * 1024),
    )(sid)

    grid2d = pl.pallas_call(
        _k5b,
        grid_spec=pltpu.PrefetchScalarGridSpec(
            num_scalar_prefetch=3,
            grid=(NSPLIT,),
            in_specs=[pl.BlockSpec((NV, 1, 128), lambda s, *_: (0, 0, 0))],
            out_specs=pl.BlockSpec((SPLIT, 1, 128), lambda s, *_: (s, 0, 0)),
        ),
        out_shape=jax.ShapeDtypeStruct((DHW, 1, 128), _f32),
        compiler_params=pltpu.CompilerParams(
            dimension_semantics=("parallel",),
            vmem_limit_bytes=100 * 1024 * 1024),
    )(idx, perm, bases, feats.reshape(NV, 1, 128))

    return grid2d.reshape(DHW, 128).T.reshape(1, 128, D, H, W)
